# Initial kernel scaffold; baseline (speedup 1.0000x reference)
import functools

import jax
import jax.numpy as jnp
from jax import lax
from jax.experimental import pallas as pl
from jax.experimental.pallas import tpu as pltpu
from jax.experimental.pallas import tpu_sc as plsc

N = 10000
D_IN = 128
D_HID = 32
D_OUT = 4
E = 320000

NSUB = 16
NCORE = 2
NWORK = NSUB * NCORE
ROWS_PER_TILE = 626
NPAD = NSUB * ROWS_PER_TILE
CHUNK = 128
CHUNKS_PER_TILE = 80
EDGES_PER_TILE = CHUNK * CHUNKS_PER_TILE
EPAD = NWORK * EDGES_PER_TILE

_mesh = plsc.VectorSubcoreMesh(core_axis_name="c", subcore_axis_name="s")


@functools.partial(
    pl.kernel,
    out_type=jax.ShapeDtypeStruct((NWORK, NPAD), jnp.float32),
    mesh=_mesh,
    scratch_types=[
        pltpu.VMEM((EDGES_PER_TILE,), jnp.int32),
        pltpu.VMEM((NPAD,), jnp.float32),
    ],
)
def _sc_degree(dst_hbm, deg_hbm, dst_v, hist_v):
    g = lax.axis_index("c") * NSUB + lax.axis_index("s")
    pltpu.sync_copy(dst_hbm.at[pl.ds(g * EDGES_PER_TILE, EDGES_PER_TILE)], dst_v)

    zeros16 = jnp.zeros((16,), jnp.float32)

    def zero_body(i, carry):
        hist_v[pl.ds(i * 16, 16)] = zeros16
        return carry

    lax.fori_loop(0, NPAD // 16, zero_body, 0)

    ones16 = jnp.ones((16,), jnp.float32)

    def acc_body(i, carry):
        idx = dst_v[pl.ds(i * 16, 16)]
        plsc.addupdate_scatter(hist_v, [idx], ones16)
        return carry

    lax.fori_loop(0, EDGES_PER_TILE // 16, acc_body, 0)
    pltpu.sync_copy(hist_v, deg_hbm.at[g])


def _make_sc_agg(feat):
    @functools.partial(
        pl.kernel,
        out_type=jax.ShapeDtypeStruct((NCORE * NPAD, feat), jnp.float32),
        mesh=_mesh,
        scratch_types=[
            pltpu.VMEM((CHUNKS_PER_TILE, CHUNK), jnp.int32),
            pltpu.VMEM((CHUNKS_PER_TILE, CHUNK), jnp.int32),
            pltpu.VMEM((CHUNK, feat), jnp.float32),
            pltpu.VMEM_SHARED((NPAD, feat), jnp.float32),
            pltpu.SemaphoreType.DMA,
        ],
    )
    def _agg(vals_hbm, src_hbm, dst_hbm, zeros_hbm, out_hbm,
             src_v, dst_v, rows_v, acc_sh, sem):
        cid = lax.axis_index("c")
        sid = lax.axis_index("s")
        g = cid * NSUB + sid

        pltpu.sync_copy(
            zeros_hbm.at[pl.ds(sid * ROWS_PER_TILE, ROWS_PER_TILE)],
            acc_sh.at[pl.ds(sid * ROWS_PER_TILE, ROWS_PER_TILE)])
        pltpu.sync_copy(src_hbm.at[pl.ds(g * CHUNKS_PER_TILE, CHUNKS_PER_TILE)], src_v)
        pltpu.sync_copy(dst_hbm.at[pl.ds(g * CHUNKS_PER_TILE, CHUNKS_PER_TILE)], dst_v)
        plsc.subcore_barrier()

        def body(j, carry):
            pltpu.async_copy(vals_hbm.at[src_v.at[j]], rows_v, sem).wait()
            pltpu.sync_copy(rows_v, acc_sh.at[dst_v.at[j]], add=True)
            return carry

        lax.fori_loop(0, CHUNKS_PER_TILE, body, 0)
        plsc.subcore_barrier()
        pltpu.sync_copy(
            acc_sh.at[pl.ds(sid * ROWS_PER_TILE, ROWS_PER_TILE)],
            out_hbm.at[pl.ds(cid * NPAD + sid * ROWS_PER_TILE, ROWS_PER_TILE)])

    return _agg


_sc_agg_hid = _make_sc_agg(D_HID)
_sc_agg_out = _make_sc_agg(2 * D_OUT)


def _tc_dinv_body(parts_ref, dinv_ref):
    deg = jnp.sum(parts_ref[...], axis=0, keepdims=True) + 1.0
    dinv_ref[...] = lax.rsqrt(deg)


_tc_dinv = pl.pallas_call(
    _tc_dinv_body,
    out_shape=jax.ShapeDtypeStruct((1, NPAD), jnp.float32))


def _tc_hs_body(x_ref, w_ref, dinv_ref, hs_ref):
    h = jnp.dot(x_ref[...], w_ref[...], preferred_element_type=jnp.float32)
    hs_ref[...] = h * dinv_ref[...]


_tc_hs = pl.pallas_call(
    _tc_hs_body,
    out_shape=jax.ShapeDtypeStruct((NPAD, D_HID), jnp.float32))


def _tc_layer2_body(acc_ref, hs_ref, dinv_ref, b1_ref, wcat_ref, zs_ref):
    agg = acc_ref[0:NPAD, :] + acc_ref[NPAD:2 * NPAD, :] + hs_ref[...]
    h = jnp.maximum(agg * dinv_ref[...] + b1_ref[...], 0.0)
    z = jnp.dot(h, wcat_ref[...], preferred_element_type=jnp.float32)
    zs_ref[...] = z * dinv_ref[...]


_tc_layer2 = pl.pallas_call(
    _tc_layer2_body,
    out_shape=jax.ShapeDtypeStruct((NPAD, 2 * D_OUT), jnp.float32))


def _tc_out_body(acc_ref, zs_ref, dinv_ref, bcat_ref, out_ref):
    agg = acc_ref[0:NPAD, :] + acc_ref[NPAD:2 * NPAD, :] + zs_ref[...]
    out_ref[...] = agg * dinv_ref[...] + bcat_ref[...]


_tc_out = pl.pallas_call(
    _tc_out_body,
    out_shape=jax.ShapeDtypeStruct((NPAD, 2 * D_OUT), jnp.float32))


def kernel(x, edge_index, W1, b1, Wmu, bmu, Wls, bls):
    src = edge_index[0]
    dst = edge_index[1]
    pad_idx = jnp.full((EPAD - E,), N, dtype=jnp.int32)
    src_p = jnp.concatenate([src, pad_idx])
    dst_p = jnp.concatenate([dst, pad_idx])
    src2d = src_p.reshape(EPAD // CHUNK, CHUNK)
    dst2d = dst_p.reshape(EPAD // CHUNK, CHUNK)
    x_p = jnp.pad(x, ((0, NPAD - N), (0, 0)))

    deg_parts = _sc_degree(dst_p)
    dinv = _tc_dinv(deg_parts).reshape(NPAD, 1)
    hs = _tc_hs(x_p, W1, dinv)

    z32 = jnp.zeros((NPAD, D_HID), jnp.float32)
    acc1 = _sc_agg_hid(hs, src2d, dst2d, z32)

    wcat = jnp.concatenate([Wmu, Wls], axis=1)
    b1r = b1.reshape(1, D_HID)
    zs = _tc_layer2(acc1, hs, dinv, b1r, wcat)

    z8 = jnp.zeros((NPAD, 2 * D_OUT), jnp.float32)
    acc2 = _sc_agg_out(zs, src2d, dst2d, z8)

    bcat = jnp.concatenate([bmu, bls]).reshape(1, 2 * D_OUT)
    out2 = _tc_out(acc2, zs, dinv, bcat)

    return (out2[:N, :D_OUT], out2[:N, D_OUT:])


# speedup vs baseline: 30.7063x; 30.7063x over previous
import functools

import jax
import jax.numpy as jnp
from jax import lax
from jax.experimental import pallas as pl
from jax.experimental.pallas import tpu as pltpu
from jax.experimental.pallas import tpu_sc as plsc

N = 10000
D_IN = 128
D_HID = 32
D_OUT = 4
E = 320000

NSUB = 16
NCORE = 2
NWORK = NSUB * NCORE
ROWS_PER_TILE = 626
NPAD = NSUB * ROWS_PER_TILE
CHUNK = 128
CHUNKS_PER_TILE = 80
EDGES_PER_TILE = CHUNK * CHUNKS_PER_TILE
EPAD = NWORK * EDGES_PER_TILE

_mesh = plsc.VectorSubcoreMesh(core_axis_name="c", subcore_axis_name="s")


@functools.partial(
    pl.kernel,
    out_type=jax.ShapeDtypeStruct((NWORK, NPAD), jnp.float32),
    mesh=_mesh,
    scratch_types=[
        pltpu.VMEM((EDGES_PER_TILE,), jnp.int32),
        pltpu.VMEM((NPAD,), jnp.float32),
    ],
    compiler_params=pltpu.CompilerParams(
        needs_layout_passes=False, use_tc_tiling_on_sc=False),
)
def _sc_degree(dst_hbm, deg_hbm, dst_v, hist_v):
    g = lax.axis_index("c") * NSUB + lax.axis_index("s")
    pltpu.sync_copy(dst_hbm.at[pl.ds(g * EDGES_PER_TILE, EDGES_PER_TILE)], dst_v)

    zeros16 = jnp.zeros((16,), jnp.float32)

    def zero_body(i, carry):
        hist_v[pl.ds(i * 16, 16)] = zeros16
        return carry

    lax.fori_loop(0, NPAD // 16, zero_body, 0)

    ones16 = jnp.ones((16,), jnp.float32)

    def acc_body(i, carry):
        idx = dst_v[pl.ds(i * 16, 16)]
        plsc.addupdate_scatter(hist_v, [idx], ones16)
        return carry

    lax.fori_loop(0, EDGES_PER_TILE // 16, acc_body, 0)
    pltpu.sync_copy(hist_v, deg_hbm.at[g])


def _make_sc_agg(feat):
    @functools.partial(
        pl.kernel,
        out_type=jax.ShapeDtypeStruct((NCORE * NPAD, feat), jnp.float32),
        mesh=_mesh,
        scratch_types=[
            pltpu.VMEM((CHUNKS_PER_TILE, CHUNK), jnp.int32),
            pltpu.VMEM((CHUNKS_PER_TILE, CHUNK), jnp.int32),
            pltpu.VMEM((CHUNK, feat), jnp.float32),
            pltpu.VMEM_SHARED((NPAD, feat), jnp.float32),
            pltpu.SemaphoreType.DMA,
        ],
        compiler_params=pltpu.CompilerParams(
            needs_layout_passes=False, use_tc_tiling_on_sc=False),
    )
    def _agg(vals_hbm, src_hbm, dst_hbm, zeros_hbm, out_hbm,
             src_v, dst_v, rows_v, acc_sh, sem):
        cid = lax.axis_index("c")
        sid = lax.axis_index("s")
        g = cid * NSUB + sid

        pltpu.sync_copy(
            zeros_hbm.at[pl.ds(sid * ROWS_PER_TILE, ROWS_PER_TILE)],
            acc_sh.at[pl.ds(sid * ROWS_PER_TILE, ROWS_PER_TILE)])
        pltpu.sync_copy(src_hbm.at[pl.ds(g * CHUNKS_PER_TILE, CHUNKS_PER_TILE)], src_v)
        pltpu.sync_copy(dst_hbm.at[pl.ds(g * CHUNKS_PER_TILE, CHUNKS_PER_TILE)], dst_v)
        plsc.subcore_barrier()

        def body(j, carry):
            pltpu.async_copy(vals_hbm.at[src_v.at[j]], rows_v, sem).wait()
            pltpu.sync_copy(rows_v, acc_sh.at[dst_v.at[j]], add=True)
            return carry

        lax.fori_loop(0, CHUNKS_PER_TILE, body, 0)
        plsc.subcore_barrier()
        pltpu.sync_copy(
            acc_sh.at[pl.ds(sid * ROWS_PER_TILE, ROWS_PER_TILE)],
            out_hbm.at[pl.ds(cid * NPAD + sid * ROWS_PER_TILE, ROWS_PER_TILE)])

    return _agg


_sc_agg_hid = _make_sc_agg(D_HID)
_sc_agg_out = _make_sc_agg(2 * D_OUT)


def _tc_dinv_body(parts_ref, dinv_ref):
    deg = jnp.sum(parts_ref[...], axis=0, keepdims=True) + 1.0
    dinv_ref[...] = lax.rsqrt(deg)


_tc_dinv = pl.pallas_call(
    _tc_dinv_body,
    out_shape=jax.ShapeDtypeStruct((1, NPAD), jnp.float32))


def _tc_hs_body(x_ref, w_ref, dinv_ref, hs_ref):
    h = jnp.dot(x_ref[...], w_ref[...], preferred_element_type=jnp.float32)
    hs_ref[...] = h * dinv_ref[...]


_tc_hs = pl.pallas_call(
    _tc_hs_body,
    out_shape=jax.ShapeDtypeStruct((NPAD, D_HID), jnp.float32))


def _tc_layer2_body(acc_ref, hs_ref, dinv_ref, b1_ref, wcat_ref, zs_ref):
    agg = acc_ref[0:NPAD, :] + acc_ref[NPAD:2 * NPAD, :] + hs_ref[...]
    h = jnp.maximum(agg * dinv_ref[...] + b1_ref[...], 0.0)
    z = jnp.dot(h, wcat_ref[...], preferred_element_type=jnp.float32)
    zs_ref[...] = z * dinv_ref[...]


_tc_layer2 = pl.pallas_call(
    _tc_layer2_body,
    out_shape=jax.ShapeDtypeStruct((NPAD, 2 * D_OUT), jnp.float32))


def _tc_out_body(acc_ref, zs_ref, dinv_ref, bcat_ref, out_ref):
    agg = acc_ref[0:NPAD, :] + acc_ref[NPAD:2 * NPAD, :] + zs_ref[...]
    out_ref[...] = agg * dinv_ref[...] + bcat_ref[...]


_tc_out = pl.pallas_call(
    _tc_out_body,
    out_shape=jax.ShapeDtypeStruct((NPAD, 2 * D_OUT), jnp.float32))


def kernel(x, edge_index, W1, b1, Wmu, bmu, Wls, bls):
    src = edge_index[0]
    dst = edge_index[1]
    pad_idx = jnp.full((EPAD - E,), N, dtype=jnp.int32)
    src_p = jnp.concatenate([src, pad_idx])
    dst_p = jnp.concatenate([dst, pad_idx])
    src2d = src_p.reshape(EPAD // CHUNK, CHUNK)
    dst2d = dst_p.reshape(EPAD // CHUNK, CHUNK)
    x_p = jnp.pad(x, ((0, NPAD - N), (0, 0)))

    deg_parts = _sc_degree(dst_p)
    dinv = _tc_dinv(deg_parts).reshape(NPAD, 1)
    hs = _tc_hs(x_p, W1, dinv)

    z32 = jnp.zeros((NPAD, D_HID), jnp.float32)
    acc1 = _sc_agg_hid(hs, src2d, dst2d, z32)

    wcat = jnp.concatenate([Wmu, Wls], axis=1)
    b1r = b1.reshape(1, D_HID)
    zs = _tc_layer2(acc1, hs, dinv, b1r, wcat)

    z8 = jnp.zeros((NPAD, 2 * D_OUT), jnp.float32)
    acc2 = _sc_agg_out(zs, src2d, dst2d, z8)

    bcat = jnp.concatenate([bmu, bls]).reshape(1, 2 * D_OUT)
    out2 = _tc_out(acc2, zs, dinv, bcat)

    return (out2[:N, :D_OUT], out2[:N, D_OUT:])


# speedup vs baseline: 39.3129x; 1.2803x over previous
import functools

import jax
import jax.numpy as jnp
from jax import lax
from jax.experimental import pallas as pl
from jax.experimental.pallas import tpu as pltpu
from jax.experimental.pallas import tpu_sc as plsc

N = 10000
D_IN = 128
D_HID = 32
D_OUT = 4
E = 320000

NSUB = 16
NCORE = 2
NWORK = NSUB * NCORE
ROWS_PER_TILE = 626
NPAD = NSUB * ROWS_PER_TILE
CHUNK = 128
CHUNKS_PER_TILE = 80
EDGES_PER_TILE = CHUNK * CHUNKS_PER_TILE
EPAD = NWORK * EDGES_PER_TILE

_mesh = plsc.VectorSubcoreMesh(core_axis_name="c", subcore_axis_name="s")


@functools.partial(
    pl.kernel,
    out_type=jax.ShapeDtypeStruct((NWORK, NPAD), jnp.float32),
    mesh=_mesh,
    scratch_types=[
        pltpu.VMEM((EDGES_PER_TILE,), jnp.int32),
        pltpu.VMEM((NPAD,), jnp.float32),
    ],
    compiler_params=pltpu.CompilerParams(
        needs_layout_passes=False, use_tc_tiling_on_sc=False),
)
def _sc_degree(dst_hbm, deg_hbm, dst_v, hist_v):
    g = lax.axis_index("c") * NSUB + lax.axis_index("s")
    pltpu.sync_copy(dst_hbm.at[pl.ds(g * EDGES_PER_TILE, EDGES_PER_TILE)], dst_v)

    zeros16 = jnp.zeros((16,), jnp.float32)

    def zero_body(i, carry):
        hist_v[pl.ds(i * 16, 16)] = zeros16
        return carry

    lax.fori_loop(0, NPAD // 16, zero_body, 0)

    ones16 = jnp.ones((16,), jnp.float32)

    def acc_body(i, carry):
        idx = dst_v[pl.ds(i * 16, 16)]
        plsc.addupdate_scatter(hist_v, [idx], ones16)
        return carry

    lax.fori_loop(0, EDGES_PER_TILE // 16, acc_body, 0)
    pltpu.sync_copy(hist_v, deg_hbm.at[g])


NBUF = 8


def _make_sc_agg(feat):
    @functools.partial(
        pl.kernel,
        out_type=jax.ShapeDtypeStruct((NCORE * NPAD, feat), jnp.float32),
        mesh=_mesh,
        scratch_types=[
            pltpu.VMEM((CHUNKS_PER_TILE, CHUNK), jnp.int32),
            pltpu.VMEM((CHUNKS_PER_TILE, CHUNK), jnp.int32),
            pltpu.VMEM((NBUF, CHUNK, feat), jnp.float32),
            pltpu.VMEM_SHARED((NPAD, feat), jnp.float32),
            pltpu.SemaphoreType.DMA((NBUF,)),
            pltpu.SemaphoreType.DMA((NBUF,)),
        ],
        compiler_params=pltpu.CompilerParams(
            needs_layout_passes=False, use_tc_tiling_on_sc=False),
    )
    def _agg(vals_hbm, src_hbm, dst_hbm, zeros_hbm, out_hbm,
             src_v, dst_v, rows_v, acc_sh, gsem, ssem):
        cid = lax.axis_index("c")
        sid = lax.axis_index("s")
        g = cid * NSUB + sid

        pltpu.sync_copy(
            zeros_hbm.at[pl.ds(sid * ROWS_PER_TILE, ROWS_PER_TILE)],
            acc_sh.at[pl.ds(sid * ROWS_PER_TILE, ROWS_PER_TILE)])
        pltpu.sync_copy(src_hbm.at[pl.ds(g * CHUNKS_PER_TILE, CHUNKS_PER_TILE)], src_v)
        pltpu.sync_copy(dst_hbm.at[pl.ds(g * CHUNKS_PER_TILE, CHUNKS_PER_TILE)], dst_v)
        plsc.subcore_barrier()

        for b in range(NBUF):
            pltpu.async_copy(vals_hbm.at[src_v.at[b]], rows_v.at[b],
                             gsem.at[b])

        def body(jj, carry):
            for b in range(NBUF):
                j = jj * NBUF + b
                pltpu.make_async_copy(vals_hbm.at[src_v.at[j]], rows_v.at[b],
                                      gsem.at[b]).wait()
                sc = pltpu.async_copy(rows_v.at[b], acc_sh.at[dst_v.at[j]],
                                      ssem.at[b], add=True)

                @pl.when(j + NBUF < CHUNKS_PER_TILE)
                def _():
                    sc.wait()
                    pltpu.async_copy(vals_hbm.at[src_v.at[j + NBUF]],
                                     rows_v.at[b], gsem.at[b])

                @pl.when(j + NBUF >= CHUNKS_PER_TILE)
                def _():
                    sc.wait()
            return carry

        lax.fori_loop(0, CHUNKS_PER_TILE // NBUF, body, 0)
        plsc.subcore_barrier()
        pltpu.sync_copy(
            acc_sh.at[pl.ds(sid * ROWS_PER_TILE, ROWS_PER_TILE)],
            out_hbm.at[pl.ds(cid * NPAD + sid * ROWS_PER_TILE, ROWS_PER_TILE)])

    return _agg


_sc_agg_hid = _make_sc_agg(D_HID)
_sc_agg_out = _make_sc_agg(2 * D_OUT)


def _tc_dinv_body(parts_ref, dinv_ref):
    deg = jnp.sum(parts_ref[...], axis=0, keepdims=True) + 1.0
    dinv_ref[...] = lax.rsqrt(deg)


_tc_dinv = pl.pallas_call(
    _tc_dinv_body,
    out_shape=jax.ShapeDtypeStruct((1, NPAD), jnp.float32))


def _tc_hs_body(x_ref, w_ref, dinv_ref, hs_ref):
    h = jnp.dot(x_ref[...], w_ref[...], preferred_element_type=jnp.float32)
    hs_ref[...] = h * dinv_ref[...]


_tc_hs = pl.pallas_call(
    _tc_hs_body,
    out_shape=jax.ShapeDtypeStruct((NPAD, D_HID), jnp.float32))


def _tc_layer2_body(acc_ref, hs_ref, dinv_ref, b1_ref, wcat_ref, zs_ref):
    agg = acc_ref[0:NPAD, :] + acc_ref[NPAD:2 * NPAD, :] + hs_ref[...]
    h = jnp.maximum(agg * dinv_ref[...] + b1_ref[...], 0.0)
    z = jnp.dot(h, wcat_ref[...], preferred_element_type=jnp.float32)
    zs_ref[...] = z * dinv_ref[...]


_tc_layer2 = pl.pallas_call(
    _tc_layer2_body,
    out_shape=jax.ShapeDtypeStruct((NPAD, 2 * D_OUT), jnp.float32))


def _tc_out_body(acc_ref, zs_ref, dinv_ref, bcat_ref, out_ref):
    agg = acc_ref[0:NPAD, :] + acc_ref[NPAD:2 * NPAD, :] + zs_ref[...]
    out_ref[...] = agg * dinv_ref[...] + bcat_ref[...]


_tc_out = pl.pallas_call(
    _tc_out_body,
    out_shape=jax.ShapeDtypeStruct((NPAD, 2 * D_OUT), jnp.float32))


def kernel(x, edge_index, W1, b1, Wmu, bmu, Wls, bls):
    src = edge_index[0]
    dst = edge_index[1]
    pad_idx = jnp.full((EPAD - E,), N, dtype=jnp.int32)
    src_p = jnp.concatenate([src, pad_idx])
    dst_p = jnp.concatenate([dst, pad_idx])
    src2d = src_p.reshape(EPAD // CHUNK, CHUNK)
    dst2d = dst_p.reshape(EPAD // CHUNK, CHUNK)
    x_p = jnp.pad(x, ((0, NPAD - N), (0, 0)))

    deg_parts = _sc_degree(dst_p)
    dinv = _tc_dinv(deg_parts).reshape(NPAD, 1)
    hs = _tc_hs(x_p, W1, dinv)

    z32 = jnp.zeros((NPAD, D_HID), jnp.float32)
    acc1 = _sc_agg_hid(hs, src2d, dst2d, z32)

    wcat = jnp.concatenate([Wmu, Wls], axis=1)
    b1r = b1.reshape(1, D_HID)
    zs = _tc_layer2(acc1, hs, dinv, b1r, wcat)

    z8 = jnp.zeros((NPAD, 2 * D_OUT), jnp.float32)
    acc2 = _sc_agg_out(zs, src2d, dst2d, z8)

    bcat = jnp.concatenate([bmu, bls]).reshape(1, 2 * D_OUT)
    out2 = _tc_out(acc2, zs, dinv, bcat)

    return (out2[:N, :D_OUT], out2[:N, D_OUT:])
